# Initial kernel scaffold; baseline (speedup 1.0000x reference)
import jax
import jax.numpy as jnp
from jax.experimental import pallas as pl


def kernel(features, adj_nei, high_att_0, diff_att_0):
    raise NotImplementedError("write your pallas kernel here")


# speedup vs baseline: 13.7272x; 13.7272x over previous
import functools

import jax
import jax.numpy as jnp
from jax import lax
from jax.experimental import pallas as pl
from jax.experimental.pallas import tpu as pltpu
from jax.experimental.pallas import tpu_sc as plsc

NODE = 10000
D = 128
E = 320000
ALPHA = 0.5
DP = 144
NC = 2
NS = 16
NW = NC * NS
EPW = E // NW
B = 80
NCHUNK = EPW // B
NP = 10240
ROWS_PER_TILE = NP // NS


def _prologue_body(f_ref, ha_ref, da_ref, gq_ref):
    F = jnp.tanh(f_ref[...])
    w = ha_ref[...] - ALPHA * da_ref[...]
    p = lax.dot_general(F, w, (((1,), (0,)), ((), ())),
                        preferred_element_type=jnp.float32)
    q = jnp.exp(p)
    gq_ref[...] = jnp.concatenate(
        [F * q, q, jnp.zeros((NODE, DP - D - 1), jnp.float32)], axis=1)


def _epilogue_body(hp_ref, o_ref):
    h = hp_ref[:NODE] + hp_ref[NP:NP + NODE]
    q = h[:, D:D + 1]
    o_ref[...] = jnp.tanh(jnp.where(q > 0, h[:, :D] / q, 0.0))


def _sc_body(gq_hbm, row_hbm, col_hbm, zero_hbm, out_hbm,
             colv, rowv, rows, acc):
    cid = lax.axis_index("c")
    sid = lax.axis_index("s")
    wid = cid * NS + sid
    zbase = sid * ROWS_PER_TILE
    pltpu.sync_copy(zero_hbm.at[pl.ds(0, ROWS_PER_TILE)],
                    acc.at[pl.ds(zbase, ROWS_PER_TILE)])
    plsc.subcore_barrier()

    base = wid * EPW

    def body(i, carry):
        off = base + i * B
        pltpu.sync_copy(col_hbm.at[pl.ds(off, B)], colv)
        pltpu.sync_copy(row_hbm.at[pl.ds(off, B)], rowv)
        pltpu.sync_copy(gq_hbm.at[colv], rows)
        pltpu.sync_copy(rows, acc.at[rowv], add=True)
        return carry

    lax.fori_loop(0, NCHUNK, body, 0)
    plsc.subcore_barrier()

    obase = cid * NP + sid * ROWS_PER_TILE
    pltpu.sync_copy(acc.at[pl.ds(zbase, ROWS_PER_TILE)],
                    out_hbm.at[pl.ds(obase, ROWS_PER_TILE)])


def kernel(features, adj_nei, high_att_0, diff_att_0):
    gq = pl.pallas_call(
        _prologue_body,
        out_shape=jax.ShapeDtypeStruct((NODE, DP), jnp.float32),
    )(features, high_att_0, diff_att_0)

    row = adj_nei[0]
    col = adj_nei[1]
    zeros = jnp.zeros((ROWS_PER_TILE, DP), jnp.float32)

    sc_fn = functools.partial(
        pl.kernel,
        mesh=plsc.VectorSubcoreMesh(core_axis_name="c", subcore_axis_name="s"),
        out_type=jax.ShapeDtypeStruct((NC * NP, DP), jnp.float32),
        scratch_types=[
            pltpu.VMEM((B,), jnp.int32),
            pltpu.VMEM((B,), jnp.int32),
            pltpu.VMEM((B, DP), jnp.float32),
            pltpu.VMEM_SHARED((NP, DP), jnp.float32),
        ],
        compiler_params=pltpu.CompilerParams(use_tc_tiling_on_sc=False),
    )(_sc_body)
    hp = sc_fn(gq, row, col, zeros)

    out = pl.pallas_call(
        _epilogue_body,
        out_shape=jax.ShapeDtypeStruct((NODE, D), jnp.float32),
    )(hp)
    return out


# speedup vs baseline: 22.4325x; 1.6342x over previous
import functools

import jax
import jax.numpy as jnp
from jax import lax
from jax.experimental import pallas as pl
from jax.experimental.pallas import tpu as pltpu
from jax.experimental.pallas import tpu_sc as plsc

NODE = 10000
D = 128
E = 320000
ALPHA = 0.5
DP = 144
NC = 2
NS = 16
NW = NC * NS
EPW = E // NW
B = 80
NCHUNK = EPW // B
NP = 10240
ROWS_PER_TILE = NP // NS


def _prologue_body(f_ref, ha_ref, da_ref, gq_ref):
    F = jnp.tanh(f_ref[...])
    w = ha_ref[...] - ALPHA * da_ref[...]
    p = lax.dot_general(F, w, (((1,), (0,)), ((), ())),
                        preferred_element_type=jnp.float32)
    q = jnp.exp(p)
    gq_ref[...] = jnp.concatenate(
        [F * q, q, jnp.zeros((NODE, DP - D - 1), jnp.float32)], axis=1)


def _epilogue_body(hp_ref, o_ref):
    h = hp_ref[:NODE] + hp_ref[NP:NP + NODE]
    q = h[:, D:D + 1]
    o_ref[...] = jnp.tanh(jnp.where(q > 0, h[:, :D] / q, 0.0))


def _sc_body(gq_hbm, row3_hbm, col3_hbm, zero_hbm, out_hbm,
             colv, rowv0, rowv1, rows0, rows1, acc, gsem, rsem):
    cid = lax.axis_index("c")
    sid = lax.axis_index("s")
    wid = cid * NS + sid
    zbase = sid * ROWS_PER_TILE
    pltpu.sync_copy(zero_hbm.at[pl.ds(0, ROWS_PER_TILE)],
                    acc.at[pl.ds(zbase, ROWS_PER_TILE)])
    pltpu.sync_copy(col3_hbm.at[wid], colv)
    plsc.subcore_barrier()

    def g_start(c, buf):
        pltpu.async_copy(gq_hbm.at[colv.at[c]], buf, gsem)

    def g_wait(c, buf):
        pltpu.make_async_copy(gq_hbm.at[colv.at[c]], buf, gsem).wait()

    def r_start(c, rbuf):
        pltpu.async_copy(row3_hbm.at[wid, c], rbuf, rsem)

    def r_wait(c, rbuf):
        pltpu.make_async_copy(row3_hbm.at[wid, c], rbuf, rsem).wait()

    def s_add(buf, rbuf):
        pltpu.sync_copy(buf, acc.at[rbuf], add=True)

    g_start(0, rows0)
    r_start(0, rowv0)

    def body(j, carry):
        c0 = 2 * j
        g_wait(c0, rows0)
        r_wait(c0, rowv0)
        g_start(c0 + 1, rows1)
        r_start(c0 + 1, rowv1)
        s_add(rows0, rowv0)
        g_wait(c0 + 1, rows1)
        r_wait(c0 + 1, rowv1)
        g_start(c0 + 2, rows0)
        r_start(c0 + 2, rowv0)
        s_add(rows1, rowv1)
        return carry

    lax.fori_loop(0, (NCHUNK - 1) // 2, body, 0)
    g_wait(NCHUNK - 1, rows0)
    r_wait(NCHUNK - 1, rowv0)
    s_add(rows0, rowv0)
    plsc.subcore_barrier()

    obase = cid * NP + sid * ROWS_PER_TILE
    pltpu.sync_copy(acc.at[pl.ds(zbase, ROWS_PER_TILE)],
                    out_hbm.at[pl.ds(obase, ROWS_PER_TILE)])


def kernel(features, adj_nei, high_att_0, diff_att_0):
    gq = pl.pallas_call(
        _prologue_body,
        out_shape=jax.ShapeDtypeStruct((NODE, DP), jnp.float32),
    )(features, high_att_0, diff_att_0)

    row3 = adj_nei[0].reshape(NW, NCHUNK, B)
    col3 = adj_nei[1].reshape(NW, NCHUNK, B)
    zeros = jnp.zeros((ROWS_PER_TILE, DP), jnp.float32)

    sc_fn = functools.partial(
        pl.kernel,
        mesh=plsc.VectorSubcoreMesh(core_axis_name="c", subcore_axis_name="s"),
        out_type=jax.ShapeDtypeStruct((NC * NP, DP), jnp.float32),
        scratch_types=[
            pltpu.VMEM((NCHUNK, B), jnp.int32),
            pltpu.VMEM((B,), jnp.int32),
            pltpu.VMEM((B,), jnp.int32),
            pltpu.VMEM((B, DP), jnp.float32),
            pltpu.VMEM((B, DP), jnp.float32),
            pltpu.VMEM_SHARED((NP, DP), jnp.float32),
            pltpu.SemaphoreType.DMA,
            pltpu.SemaphoreType.DMA,
        ],
        compiler_params=pltpu.CompilerParams(use_tc_tiling_on_sc=False),
    )(_sc_body)
    hp = sc_fn(gq, row3, col3, zeros)

    out = pl.pallas_call(
        _epilogue_body,
        out_shape=jax.ShapeDtypeStruct((NODE, D), jnp.float32),
    )(hp)
    return out
